# Initial kernel scaffold; baseline (speedup 1.0000x reference)
import jax
import jax.numpy as jnp
from jax.experimental import pallas as pl


def kernel(x_loc, x_feat, W_rel, b_rel, W_root):
    raise NotImplementedError("write your pallas kernel here")


# speedup vs baseline: 20.2939x; 20.2939x over previous
import jax
import jax.numpy as jnp
from jax.experimental import pallas as pl

_B, _N, _K, _CIN, _COUT = 4, 4096, 16, 64, 64
_FAN = _CIN + 3
_R = 256


def _knn_conv_kernel(loc_row_ref, loc_all_ref, gt_all_ref, gt_row_ref,
                     wrel_ref, brel_ref, wroot_ref, out_ref):
    i = pl.program_id(1)
    lrow = loc_row_ref[0]
    lcol = loc_all_ref[0]

    dx = lrow[0][:, None] - lcol[0][None, :]
    dy = lrow[1][:, None] - lcol[1][None, :]
    dz = lrow[2][:, None] - lcol[2][None, :]
    d = dx * dx + dy * dy + dz * dz

    row_g = jax.lax.broadcasted_iota(jnp.int32, (_R, _N), 0) + i * _R
    col_g = jax.lax.broadcasted_iota(jnp.int32, (_R, _N), 1)
    d = jnp.where(row_g == col_g, jnp.inf, d)

    m_acc = jnp.zeros((_R, _N), jnp.float32)
    for _ in range(_K):
        row_min = jnp.min(d, axis=1, keepdims=True)
        sel = d == row_min
        m_acc = m_acc + sel.astype(jnp.float32)
        d = jnp.where(sel, jnp.inf, d)

    gt = gt_all_ref[0]
    aggr = jnp.dot(m_acc, gt, preferred_element_type=jnp.float32)
    feat_row = gt_row_ref[0]
    out = (jnp.dot(aggr, wrel_ref[...], preferred_element_type=jnp.float32)
           + brel_ref[0][None, :]
           + jnp.dot(feat_row, wroot_ref[...], preferred_element_type=jnp.float32))
    out_ref[0] = jnp.maximum(out, 0.0)


@jax.jit
def kernel(x_loc, x_feat, W_rel, b_rel, W_root):
    gt = jnp.concatenate([x_loc, x_feat], axis=1).transpose(0, 2, 1)
    brel2 = b_rel.reshape(1, _COUT)

    out_nk = pl.pallas_call(
        _knn_conv_kernel,
        grid=(_B, _N // _R),
        in_specs=[
            pl.BlockSpec((1, 3, _R), lambda b, i: (b, 0, i)),
            pl.BlockSpec((1, 3, _N), lambda b, i: (b, 0, 0)),
            pl.BlockSpec((1, _N, _FAN), lambda b, i: (b, 0, 0)),
            pl.BlockSpec((1, _R, _FAN), lambda b, i: (b, i, 0)),
            pl.BlockSpec((_FAN, _COUT), lambda b, i: (0, 0)),
            pl.BlockSpec((1, _COUT), lambda b, i: (0, 0)),
            pl.BlockSpec((_FAN, _COUT), lambda b, i: (0, 0)),
        ],
        out_specs=pl.BlockSpec((1, _R, _COUT), lambda b, i: (b, i, 0)),
        out_shape=jax.ShapeDtypeStruct((_B, _N, _COUT), jnp.float32),
    )(x_loc, x_loc, gt, gt, W_rel, brel2, W_root)

    return (x_loc, out_nk.transpose(0, 2, 1))


# speedup vs baseline: 44.8945x; 2.2122x over previous
import jax
import jax.numpy as jnp
from jax.experimental import pallas as pl

_B, _N, _K, _CIN, _COUT = 4, 4096, 16, 64, 64
_FAN = _CIN + 3
_R = 128
_CW = 128
_NC = _N // _CW
_T = 4


def _knn_conv_kernel(loc_row_ref, loc_all_ref, gt_all_ref, gt_row_ref,
                     wrel_ref, brel_ref, wroot_ref, out_ref):
    i = pl.program_id(1)
    lrow = loc_row_ref[0]
    lcol = loc_all_ref[0]
    ax = lrow[0][:, None]
    ay = lrow[1][:, None]
    az = lrow[2][:, None]
    row_g = i * _R + jax.lax.broadcasted_iota(jnp.int32, (_R, _CW), 0)

    inf = jnp.inf
    top = [jnp.full((_R, _CW), inf, jnp.float32) for _ in range(_T)]
    chunks = []
    for c in range(_NC):
        sl = slice(c * _CW, (c + 1) * _CW)
        dx = ax - lcol[0, sl][None, :]
        dy = ay - lcol[1, sl][None, :]
        dz = az - lcol[2, sl][None, :]
        v = dx * dx + dy * dy + dz * dz
        col_g = c * _CW + jax.lax.broadcasted_iota(jnp.int32, (_R, _CW), 1)
        v = jnp.where(row_g == col_g, inf, v)
        chunks.append(v)
        for s in range(_T):
            lo = jnp.minimum(top[s], v)
            v = jnp.maximum(top[s], v)
            top[s] = lo
    d = jnp.concatenate(chunks, axis=1)

    u = jnp.concatenate(top, axis=1)
    tstar = None
    for _ in range(_K):
        tstar = jnp.min(u, axis=1, keepdims=True)
        u = jnp.where(u == tstar, inf, u)

    m_sel = (d <= tstar).astype(jnp.float32)

    gt = gt_all_ref[0]
    feat_row = gt_row_ref[0]
    wrel = wrel_ref[...]
    wroot = wroot_ref[...]
    brel = brel_ref[0][None, :]

    aggr = jnp.dot(m_sel, gt, preferred_element_type=jnp.float32)
    out = (jnp.dot(aggr, wrel, preferred_element_type=jnp.float32) + brel
           + jnp.dot(feat_row, wroot, preferred_element_type=jnp.float32))
    out_ref[0] = jnp.maximum(out, 0.0)

    bad = jnp.any(aggr[:, _FAN] != jnp.float32(_K))

    @pl.when(bad)
    def _fallback():
        dd = d
        m_acc = jnp.zeros((_R, _N), jnp.float32)
        for _ in range(_K):
            row_min = jnp.min(dd, axis=1, keepdims=True)
            sel = dd == row_min
            m_acc = m_acc + sel.astype(jnp.float32)
            dd = jnp.where(sel, inf, dd)
        a2 = jnp.dot(m_acc, gt, preferred_element_type=jnp.float32)
        o2 = (jnp.dot(a2, wrel, preferred_element_type=jnp.float32) + brel
              + jnp.dot(feat_row, wroot, preferred_element_type=jnp.float32))
        out_ref[0] = jnp.maximum(o2, 0.0)


@jax.jit
def kernel(x_loc, x_feat, W_rel, b_rel, W_root):
    gt = jnp.concatenate(
        [x_loc, x_feat, jnp.ones((_B, 1, _N), jnp.float32)], axis=1
    ).transpose(0, 2, 1)
    zrow = jnp.zeros((1, _COUT), jnp.float32)
    wrel_p = jnp.concatenate([W_rel, zrow], axis=0)
    wroot_p = jnp.concatenate([W_root, zrow], axis=0)
    brel2 = b_rel.reshape(1, _COUT)
    fp = _FAN + 1

    out_nk = pl.pallas_call(
        _knn_conv_kernel,
        grid=(_B, _N // _R),
        in_specs=[
            pl.BlockSpec((1, 3, _R), lambda b, i: (b, 0, i)),
            pl.BlockSpec((1, 3, _N), lambda b, i: (b, 0, 0)),
            pl.BlockSpec((1, _N, fp), lambda b, i: (b, 0, 0)),
            pl.BlockSpec((1, _R, fp), lambda b, i: (b, i, 0)),
            pl.BlockSpec((fp, _COUT), lambda b, i: (0, 0)),
            pl.BlockSpec((1, _COUT), lambda b, i: (0, 0)),
            pl.BlockSpec((fp, _COUT), lambda b, i: (0, 0)),
        ],
        out_specs=pl.BlockSpec((1, _R, _COUT), lambda b, i: (b, i, 0)),
        out_shape=jax.ShapeDtypeStruct((_B, _N, _COUT), jnp.float32),
    )(x_loc, x_loc, gt, gt, wrel_p, brel2, wroot_p)

    return (x_loc, out_nk.transpose(0, 2, 1))


# speedup vs baseline: 45.5076x; 1.0137x over previous
import jax
import jax.numpy as jnp
from jax.experimental import pallas as pl
from jax.experimental.pallas import tpu as pltpu

_B, _N, _K, _CIN, _COUT = 4, 4096, 16, 64, 64
_FAN = _CIN + 3
_R = 128
_CW = 128
_NC = _N // _CW
_T = 4


def _knn_conv_kernel(loc_row_ref, loc_all_ref, gt_all_ref, gt_row_ref,
                     wrel_ref, brel_ref, wroot_ref, out_ref, d_ref):
    i = pl.program_id(1)
    lrow = loc_row_ref[0]
    lcol = loc_all_ref[0]
    ax = lrow[0][:, None]
    ay = lrow[1][:, None]
    az = lrow[2][:, None]

    inf = jnp.inf
    top = [jnp.full((_R, _CW), inf, jnp.float32) for _ in range(_T)]
    for c in range(_NC):
        sl = slice(c * _CW, (c + 1) * _CW)
        dx = ax - lcol[0, sl][None, :]
        dy = ay - lcol[1, sl][None, :]
        dz = az - lcol[2, sl][None, :]
        v = dx * dx + dy * dy + dz * dz
        d_ref[:, sl] = v
        for s in range(_T):
            lo = jnp.minimum(top[s], v)
            v = jnp.maximum(top[s], v)
            top[s] = lo

    u = jnp.concatenate(top, axis=1)
    tstar = None
    for _ in range(_K + 1):
        tstar = jnp.min(u, axis=1, keepdims=True)
        u = jnp.where(u == tstar, inf, u)

    m_sel = (d_ref[...] <= tstar).astype(jnp.float32)

    gt = gt_all_ref[0]
    feat_row = gt_row_ref[0]
    wrel = wrel_ref[...]
    wroot = wroot_ref[...]
    brel = brel_ref[0][None, :]

    aggr = jnp.dot(m_sel, gt, preferred_element_type=jnp.float32)
    out = (jnp.dot(aggr, wrel, preferred_element_type=jnp.float32) + brel
           + jnp.dot(feat_row, wroot - wrel, preferred_element_type=jnp.float32))
    out_ref[0] = jnp.maximum(out, 0.0)

    bad = jnp.any(aggr[:, _FAN] != jnp.float32(_K + 1))

    @pl.when(bad)
    def _fallback():
        row_g = i * _R + jax.lax.broadcasted_iota(jnp.int32, (_R, _N), 0)
        col_g = jax.lax.broadcasted_iota(jnp.int32, (_R, _N), 1)
        dd = jnp.where(row_g == col_g, inf, d_ref[...])
        m_acc = jnp.zeros((_R, _N), jnp.float32)
        for _ in range(_K):
            row_min = jnp.min(dd, axis=1, keepdims=True)
            sel = dd == row_min
            m_acc = m_acc + sel.astype(jnp.float32)
            dd = jnp.where(sel, inf, dd)
        a2 = jnp.dot(m_acc, gt, preferred_element_type=jnp.float32)
        o2 = (jnp.dot(a2, wrel, preferred_element_type=jnp.float32) + brel
              + jnp.dot(feat_row, wroot, preferred_element_type=jnp.float32))
        out_ref[0] = jnp.maximum(o2, 0.0)


@jax.jit
def kernel(x_loc, x_feat, W_rel, b_rel, W_root):
    gt = jnp.concatenate(
        [x_loc, x_feat, jnp.ones((_B, 1, _N), jnp.float32)], axis=1
    ).transpose(0, 2, 1)
    zrow = jnp.zeros((1, _COUT), jnp.float32)
    wrel_p = jnp.concatenate([W_rel, zrow], axis=0)
    wroot_p = jnp.concatenate([W_root, zrow], axis=0)
    brel2 = b_rel.reshape(1, _COUT)
    fp = _FAN + 1

    out_nk = pl.pallas_call(
        _knn_conv_kernel,
        grid=(_B, _N // _R),
        in_specs=[
            pl.BlockSpec((1, 3, _R), lambda b, i: (b, 0, i)),
            pl.BlockSpec((1, 3, _N), lambda b, i: (b, 0, 0)),
            pl.BlockSpec((1, _N, fp), lambda b, i: (b, 0, 0)),
            pl.BlockSpec((1, _R, fp), lambda b, i: (b, i, 0)),
            pl.BlockSpec((fp, _COUT), lambda b, i: (0, 0)),
            pl.BlockSpec((1, _COUT), lambda b, i: (0, 0)),
            pl.BlockSpec((fp, _COUT), lambda b, i: (0, 0)),
        ],
        out_specs=pl.BlockSpec((1, _R, _COUT), lambda b, i: (b, i, 0)),
        out_shape=jax.ShapeDtypeStruct((_B, _N, _COUT), jnp.float32),
        scratch_shapes=[pltpu.VMEM((_R, _N), jnp.float32)],
    )(x_loc, x_loc, gt, gt, wrel_p, brel2, wroot_p)

    return (x_loc, out_nk.transpose(0, 2, 1))
